# baseline (speedup 1.0000x reference)
import jax
import jax.numpy as jnp
from jax.experimental import pallas as pl

_IB = 4


def _mlp_kernel(s_ref, w1_ref, b1_ref, w2t_ref, b2_ref, o_ref):
    H = b1_ref.shape[0]
    s = s_ref[0]
    n = s.shape[0]
    nb = n // _IB
    i0 = pl.program_id(1) * nb
    W1a = w1_ref[:H, :]
    W1b = w1_ref[H:, :]
    s_blk = s_ref[0, pl.ds(i0, nb), :]
    AT = jax.lax.dot_general(W1a, s_blk, (((0,), (1,)), ((), ())),
                             preferred_element_type=jnp.float32) + b1_ref[...]
    BT = jax.lax.dot_general(W1b, s, (((0,), (1,)), ((), ())),
                             preferred_element_type=jnp.float32)
    w2t = w2t_ref[...].astype(jnp.bfloat16)
    b2v = b2_ref[0, 0]

    ATb = AT.astype(jnp.bfloat16)
    BTb = BT.astype(jnp.bfloat16)
    zero = jnp.zeros((), jnp.bfloat16)

    for i in range(nb):
        col = ATb[:, i:i + 1]
        m = jnp.maximum(BTb + col, zero)
        row = jax.lax.dot_general(w2t, m, (((1,), (0,)), ((), ())),
                                  preferred_element_type=jnp.float32)
        row = jax.nn.sigmoid(row + b2v)
        mask = (s_blk[i:i + 1, :] != 0).astype(jnp.float32)
        o_ref[0, i:i + 1, :] = row * mask


def kernel(structure, W1, b1, W2, b2):
    Bn, N, K = structure.shape
    H = b1.shape[0]
    b1c = b1.reshape(H, 1)
    b2c = b2.reshape(1, 1)
    w2t = W2.reshape(1, H)
    out = pl.pallas_call(
        _mlp_kernel,
        grid=(Bn, _IB),
        in_specs=[
            pl.BlockSpec((1, N, K), lambda b, t: (b, 0, 0)),
            pl.BlockSpec((2 * H, H), lambda b, t: (0, 0)),
            pl.BlockSpec((H, 1), lambda b, t: (0, 0)),
            pl.BlockSpec((1, H), lambda b, t: (0, 0)),
            pl.BlockSpec((1, 1), lambda b, t: (0, 0)),
        ],
        out_specs=pl.BlockSpec((1, N // _IB, N), lambda b, t: (b, t, 0)),
        out_shape=jax.ShapeDtypeStruct((Bn, N, N), jnp.float32),
    )(structure, W1, b1c, w2t, b2c)
    return out


# speedup vs baseline: 1.2721x; 1.2721x over previous
import jax
import jax.numpy as jnp
from jax.experimental import pallas as pl


def _mlp_kernel(s_ref, w1_ref, b1_ref, w2t_ref, b2_ref, o_ref):
    H = b1_ref.shape[0]
    Bn, n, _ = s_ref.shape
    W1a = w1_ref[:H, :]
    W1b = w1_ref[H:, :]
    w2t = w2t_ref[...].astype(jnp.bfloat16)
    b2v = b2_ref[0, 0]
    zero = jnp.zeros((), jnp.bfloat16)

    for b in range(Bn):
        s = s_ref[b]
        AT = jax.lax.dot_general(W1a, s, (((0,), (1,)), ((), ())),
                                 preferred_element_type=jnp.float32) + b1_ref[...]
        BT = jax.lax.dot_general(W1b, s, (((0,), (1,)), ((), ())),
                                 preferred_element_type=jnp.float32)
        ATb = AT.astype(jnp.bfloat16)
        BTb = BT.astype(jnp.bfloat16)
        for i in range(n):
            col = ATb[:, i:i + 1]
            m = jnp.maximum(BTb + col, zero)
            row = jax.lax.dot_general(w2t, m, (((1,), (0,)), ((), ())),
                                      preferred_element_type=jnp.float32)
            row = jax.nn.sigmoid(row + b2v)
            mask = (s[i:i + 1, :] != 0).astype(jnp.float32)
            o_ref[b, i:i + 1, :] = row * mask


def kernel(structure, W1, b1, W2, b2):
    Bn, N, K = structure.shape
    H = b1.shape[0]
    b1c = b1.reshape(H, 1)
    b2c = b2.reshape(1, 1)
    w2t = W2.reshape(1, H)
    out = pl.pallas_call(
        _mlp_kernel,
        grid=(1,),
        in_specs=[
            pl.BlockSpec((Bn, N, K), lambda t: (0, 0, 0)),
            pl.BlockSpec((2 * H, H), lambda t: (0, 0)),
            pl.BlockSpec((H, 1), lambda t: (0, 0)),
            pl.BlockSpec((1, H), lambda t: (0, 0)),
            pl.BlockSpec((1, 1), lambda t: (0, 0)),
        ],
        out_specs=pl.BlockSpec((Bn, N, N), lambda t: (0, 0, 0)),
        out_shape=jax.ShapeDtypeStruct((Bn, N, N), jnp.float32),
    )(structure, W1, b1c, w2t, b2c)
    return out
